# Initial kernel scaffold; baseline (speedup 1.0000x reference)
import functools

import jax
import jax.numpy as jnp
from jax import lax
from jax.experimental import pallas as pl
from jax.experimental.pallas import tpu as pltpu
from jax.experimental.pallas import tpu_sc as plsc

N = 10000
E = 320000
D_IN = 128
D_HID = 128
D_OUT = 16

K = 125
NROWS = E // K
NC, NS = 2, 16
NW = NC * NS
ROWS_W = NROWS // NW
RPT = N // NS


def _build_agg(D, with_deg):
  mesh = plsc.VectorSubcoreMesh(core_axis_name="c", subcore_axis_name="s")

  out_type = [jax.ShapeDtypeStruct((N, D), jnp.float32),
              jax.ShapeDtypeStruct((N, D), jnp.float32)]
  if with_deg:
    out_type += [jax.ShapeDtypeStruct((N,), jnp.float32),
                 jax.ShapeDtypeStruct((N,), jnp.float32)]

  scratch = [
      pltpu.VMEM_SHARED((N, D), jnp.float32),
      pltpu.VMEM((ROWS_W, K), jnp.int32),
      pltpu.VMEM((ROWS_W, K), jnp.int32),
      pltpu.VMEM((K, D), jnp.float32),
      pltpu.VMEM((K, D), jnp.float32),
  ]
  if with_deg:
    scratch += [
        pltpu.VMEM_SHARED((N,), jnp.float32),
        pltpu.VMEM((128,), jnp.float32),
        pltpu.VMEM((640,), jnp.float32),
    ]

  def body(x_hbm, srcm, dstm, *rest):
    if with_deg:
      (agg_a, agg_b, deg_a, deg_b, agg_sh, src_v, dst_v, gbuf, zbuf,
       deg_sh, ones_v, zb1) = rest
    else:
      agg_a, agg_b, agg_sh, src_v, dst_v, gbuf, zbuf = rest

    cid = lax.axis_index("c")
    sid = lax.axis_index("s")
    wid = cid * NS + sid

    zeros16 = jnp.zeros((16,), jnp.float32)

    def zstore(t, _):
      i = t // (D // 16)
      j = t % (D // 16)
      zbuf[i, pl.ds(j * 16, 16)] = zeros16
      return ()
    lax.fori_loop(0, K * (D // 16), zstore, ())
    for q in range(RPT // K):
      pltpu.sync_copy(zbuf, agg_sh.at[pl.ds(sid * RPT + q * K, K)])

    if with_deg:
      def zstore1(t, _):
        zb1[pl.ds(t * 16, 16)] = zeros16
        return ()
      lax.fori_loop(0, 640 // 16, zstore1, ())
      def ostore(t, _):
        ones_v[pl.ds(t * 16, 16)] = jnp.full((16,), 1.0, jnp.float32)
        return ()
      lax.fori_loop(0, 128 // 16, ostore, ())
      @pl.when(sid < 15)
      def _():
        pltpu.sync_copy(zb1, deg_sh.at[pl.ds(sid * 640, 640)])
      @pl.when(sid == 15)
      def _():
        pltpu.sync_copy(zb1.at[pl.ds(0, 400)], deg_sh.at[pl.ds(9600, 400)])

    base = wid * ROWS_W
    pltpu.sync_copy(srcm.at[pl.ds(base, ROWS_W)], src_v)
    pltpu.sync_copy(dstm.at[pl.ds(base, ROWS_W)], dst_v)

    plsc.subcore_barrier()

    def chunk(j, _):
      pltpu.sync_copy(x_hbm.at[src_v.at[j]], gbuf)
      pltpu.sync_copy(gbuf, agg_sh.at[dst_v.at[j]], add=True)
      if with_deg:
        pltpu.sync_copy(ones_v.at[pl.ds(0, K)], deg_sh.at[dst_v.at[j]],
                        add=True)
      return ()
    lax.fori_loop(0, ROWS_W, chunk, ())

    plsc.subcore_barrier()

    r0 = sid * RPT
    @pl.when(cid == 0)
    def _():
      pltpu.sync_copy(agg_sh.at[pl.ds(r0, RPT)], agg_a.at[pl.ds(r0, RPT)])
    @pl.when(cid == 1)
    def _():
      pltpu.sync_copy(agg_sh.at[pl.ds(r0, RPT)], agg_b.at[pl.ds(r0, RPT)])
    if with_deg:
      for c, deg_o in ((0, deg_a), (1, deg_b)):
        @pl.when(jnp.logical_and(cid == c, sid < 15))
        def _(deg_o=deg_o):
          pltpu.sync_copy(deg_sh.at[pl.ds(sid * 640, 640)],
                          deg_o.at[pl.ds(sid * 640, 640)])
        @pl.when(jnp.logical_and(cid == c, sid == 15))
        def _(deg_o=deg_o):
          pltpu.sync_copy(deg_sh.at[pl.ds(9600, 400)],
                          deg_o.at[pl.ds(9600, 400)])

  return pl.kernel(body, out_type=out_type, mesh=mesh,
                   scratch_types=scratch)


_agg128_deg = _build_agg(D_HID, with_deg=True)
_agg128 = _build_agg(D_HID, with_deg=False)
_agg16 = _build_agg(D_OUT, with_deg=False)


def _dense_relu(agg_a, agg_b, deg_a, deg_b, W, b):
  def body(aa, ab, da, db, w, bb, out):
    dinv = 1.0 / jnp.maximum(da[...] + db[...], 1.0)
    acc = (aa[...] + ab[...]) * dinv
    out[...] = jnp.maximum(
        jnp.dot(acc, w[...], preferred_element_type=jnp.float32) + bb[...],
        0.0)
  return pl.pallas_call(
      body,
      out_shape=jax.ShapeDtypeStruct((N, W.shape[1]), jnp.float32),
  )(agg_a, agg_b, deg_a, deg_b, W, b)


def _dense_relu_proj(agg_a, agg_b, deg_a, deg_b, W, b, Wn):
  def body(aa, ab, da, db, w, bb, wn, out):
    dinv = 1.0 / jnp.maximum(da[...] + db[...], 1.0)
    acc = (aa[...] + ab[...]) * dinv
    h = jnp.maximum(
        jnp.dot(acc, w[...], preferred_element_type=jnp.float32) + bb[...],
        0.0)
    out[...] = jnp.dot(h, wn[...], preferred_element_type=jnp.float32)
  return pl.pallas_call(
      body,
      out_shape=jax.ShapeDtypeStruct((N, Wn.shape[1]), jnp.float32),
  )(agg_a, agg_b, deg_a, deg_b, W, b, Wn)


def _final_affine(agg_a, agg_b, deg_a, deg_b, b):
  def body(aa, ab, da, db, bb, out):
    dinv = 1.0 / jnp.maximum(da[...] + db[...], 1.0)
    out[...] = (aa[...] + ab[...]) * dinv + bb[...]
  return pl.pallas_call(
      body,
      out_shape=jax.ShapeDtypeStruct((N, D_OUT), jnp.float32),
  )(agg_a, agg_b, deg_a, deg_b, b)


def kernel(nodes, edge_index, W1, b1, W2, b2, W3, b3):
  ei = edge_index.astype(jnp.int32)
  srcm = ei[0].reshape(NROWS, K)
  dstm = ei[1].reshape(NROWS, K)
  b1r = b1.reshape(1, D_HID)
  b2r = b2.reshape(1, D_HID)
  b3r = b3.reshape(1, D_OUT)

  a1a, a1b, dega, degb = _agg128_deg(nodes, srcm, dstm)
  dega = dega.reshape(N, 1)
  degb = degb.reshape(N, 1)
  h1 = _dense_relu(a1a, a1b, dega, degb, W1, b1r)
  a2a, a2b = _agg128(h1, srcm, dstm)
  z = _dense_relu_proj(a2a, a2b, dega, degb, W2, b2r, W3)
  a3a, a3b = _agg16(z, srcm, dstm)
  return _final_affine(a3a, a3b, dega, degb, b3r)


# speedup vs baseline: 8.7478x; 8.7478x over previous
import functools

import jax
import jax.numpy as jnp
from jax import lax
from jax.experimental import pallas as pl
from jax.experimental.pallas import tpu as pltpu
from jax.experimental.pallas import tpu_sc as plsc

N = 10000
E = 320000
D_IN = 128
D_HID = 128
D_OUT = 16

K = 125
NROWS = E // K
NC, NS = 2, 16
NW = NC * NS
ROWS_W = NROWS // NW
ZR = 640
ZLAST = N - 15 * ZR


def _per_tile_strip(sid, fn):
  @pl.when(sid < 15)
  def _():
    fn(sid * ZR, ZR)
  @pl.when(sid == 15)
  def _():
    fn(15 * ZR, ZLAST)


def _build_agg(D, with_deg):
  mesh = plsc.VectorSubcoreMesh(core_axis_name="c", subcore_axis_name="s")

  out_type = [jax.ShapeDtypeStruct((N, D), jnp.float32),
              jax.ShapeDtypeStruct((N, D), jnp.float32)]
  if with_deg:
    out_type += [jax.ShapeDtypeStruct((N,), jnp.float32),
                 jax.ShapeDtypeStruct((N,), jnp.float32)]

  scratch = [
      pltpu.VMEM_SHARED((N, D), jnp.float32),
      pltpu.VMEM((ROWS_W, K), jnp.int32),
      pltpu.VMEM((ROWS_W, K), jnp.int32),
      pltpu.VMEM((K, D), jnp.float32),
      pltpu.VMEM((16, D), jnp.float32),
  ]
  if with_deg:
    scratch += [
        pltpu.VMEM_SHARED((N,), jnp.float32),
        pltpu.VMEM((128,), jnp.float32),
        pltpu.VMEM((640,), jnp.float32),
    ]

  def body(x_hbm, srcm, dstm, *rest):
    if with_deg:
      (agg_a, agg_b, deg_a, deg_b, agg_sh, src_v, dst_v, gbuf, zbuf,
       deg_sh, ones_v, zb1) = rest
    else:
      agg_a, agg_b, agg_sh, src_v, dst_v, gbuf, zbuf = rest

    cid = lax.axis_index("c")
    sid = lax.axis_index("s")
    wid = cid * NS + sid

    zeros16 = jnp.zeros((16,), jnp.float32)

    def zstore(t, _):
      i = t // (D // 16)
      j = t % (D // 16)
      zbuf[i, pl.ds(j * 16, 16)] = zeros16
      return ()
    lax.fori_loop(0, 16 * (D // 16), zstore, ())

    def zero_strip(base, nrows):
      def zcopy(q, _):
        pltpu.sync_copy(zbuf, agg_sh.at[pl.ds(base + q * 16, 16)])
        return ()
      lax.fori_loop(0, nrows // 16, zcopy, ())
    _per_tile_strip(sid, zero_strip)

    if with_deg:
      def zstore1(t, _):
        zb1[pl.ds(t * 16, 16)] = zeros16
        return ()
      lax.fori_loop(0, 640 // 16, zstore1, ())
      def ostore(t, _):
        ones_v[pl.ds(t * 16, 16)] = jnp.full((16,), 1.0, jnp.float32)
        return ()
      lax.fori_loop(0, 128 // 16, ostore, ())
      @pl.when(sid < 15)
      def _():
        pltpu.sync_copy(zb1, deg_sh.at[pl.ds(sid * 640, 640)])
      @pl.when(sid == 15)
      def _():
        pltpu.sync_copy(zb1.at[pl.ds(0, 400)], deg_sh.at[pl.ds(9600, 400)])

    base = wid * ROWS_W
    pltpu.sync_copy(srcm.at[pl.ds(base, ROWS_W)], src_v)
    pltpu.sync_copy(dstm.at[pl.ds(base, ROWS_W)], dst_v)

    plsc.subcore_barrier()

    def chunk(j, _):
      pltpu.sync_copy(x_hbm.at[src_v.at[j]], gbuf)
      pltpu.sync_copy(gbuf, agg_sh.at[dst_v.at[j]], add=True)
      if with_deg:
        pltpu.sync_copy(ones_v.at[pl.ds(0, K)], deg_sh.at[dst_v.at[j]],
                        add=True)
      return ()
    lax.fori_loop(0, ROWS_W, chunk, ())

    plsc.subcore_barrier()

    for c, agg_o in ((0, agg_a), (1, agg_b)):
      @pl.when(cid == c)
      def _(agg_o=agg_o):
        def drain_strip(base, nrows):
          pltpu.sync_copy(agg_sh.at[pl.ds(base, nrows)],
                          agg_o.at[pl.ds(base, nrows)])
        _per_tile_strip(sid, drain_strip)
    if with_deg:
      for c, deg_o in ((0, deg_a), (1, deg_b)):
        @pl.when(cid == c)
        def _(deg_o=deg_o):
          def deg_strip(base, nrows):
            pltpu.sync_copy(deg_sh.at[pl.ds(base, nrows)],
                            zb1.at[pl.ds(0, nrows)])
            pltpu.sync_copy(zb1.at[pl.ds(0, nrows)],
                            deg_o.at[pl.ds(base, nrows)])
          _per_tile_strip(sid, deg_strip)

  return pl.kernel(body, out_type=out_type, mesh=mesh,
                   scratch_types=scratch)


_agg128_deg = _build_agg(D_HID, with_deg=True)
_agg128 = _build_agg(D_HID, with_deg=False)


def _dense_relu(agg_a, agg_b, deg_a, deg_b, W, b):
  def body(aa, ab, da, db, w, bb, out):
    dinv = 1.0 / jnp.maximum(da[...] + db[...], 1.0)
    acc = (aa[...] + ab[...]) * dinv
    out[...] = jnp.maximum(
        jnp.dot(acc, w[...], preferred_element_type=jnp.float32) + bb[...],
        0.0)
  return pl.pallas_call(
      body,
      out_shape=jax.ShapeDtypeStruct((N, W.shape[1]), jnp.float32),
  )(agg_a, agg_b, deg_a, deg_b, W, b)


def _dense_final(agg_a, agg_b, deg_a, deg_b, W, b):
  def body(aa, ab, da, db, w, bb, out):
    dinv = 1.0 / jnp.maximum(da[...] + db[...], 1.0)
    acc = (aa[...] + ab[...]) * dinv
    out[...] = jnp.dot(acc, w[...], preferred_element_type=jnp.float32) + bb[...]
  return pl.pallas_call(
      body,
      out_shape=jax.ShapeDtypeStruct((N, W.shape[1]), jnp.float32),
  )(agg_a, agg_b, deg_a, deg_b, W, b)


def kernel(nodes, edge_index, W1, b1, W2, b2, W3, b3):
  ei = edge_index.astype(jnp.int32)
  srcm = ei[0].reshape(NROWS, K)
  dstm = ei[1].reshape(NROWS, K)
  b1r = b1.reshape(1, D_HID)
  b2r = b2.reshape(1, D_HID)
  b3r = b3.reshape(1, D_OUT)

  a1a, a1b, dega, degb = _agg128_deg(nodes, srcm, dstm)
  dega = dega.reshape(N, 1)
  degb = degb.reshape(N, 1)
  h1 = _dense_relu(a1a, a1b, dega, degb, W1, b1r)
  a2a, a2b = _agg128(h1, srcm, dstm)
  h2 = _dense_relu(a2a, a2b, dega, degb, W2, b2r)
  a3a, a3b = _agg128(h2, srcm, dstm)
  return _dense_final(a3a, a3b, dega, degb, W3, b3r)


# speedup vs baseline: 10.8310x; 1.2381x over previous
import functools

import jax
import jax.numpy as jnp
from jax import lax
from jax.experimental import pallas as pl
from jax.experimental.pallas import tpu as pltpu
from jax.experimental.pallas import tpu_sc as plsc

N = 10000
E = 320000
D_IN = 128
D_HID = 128
D_OUT = 16

K = 125
NROWS = E // K
NC, NS = 2, 16
NW = NC * NS
ROWS_W = NROWS // NW
HR = ROWS_W // 2
ZR = 640
ZLAST = N - 15 * ZR


def _per_tile_strip(sid, fn):
  @pl.when(sid < 15)
  def _():
    fn(sid * ZR, ZR)
  @pl.when(sid == 15)
  def _():
    fn(15 * ZR, ZLAST)


def _build_agg(D, with_deg):
  mesh = plsc.VectorSubcoreMesh(core_axis_name="c", subcore_axis_name="s")

  out_type = [jax.ShapeDtypeStruct((N, D), jnp.float32),
              jax.ShapeDtypeStruct((N, D), jnp.float32)]
  if with_deg:
    out_type += [jax.ShapeDtypeStruct((N,), jnp.float32),
                 jax.ShapeDtypeStruct((N,), jnp.float32)]

  scratch = [
      pltpu.VMEM_SHARED((N, D), jnp.float32),
      pltpu.VMEM((HR, K), jnp.int32),
      pltpu.VMEM((HR, K), jnp.int32),
      pltpu.VMEM((K, D), jnp.float32),
      pltpu.VMEM((K, D), jnp.float32),
      pltpu.VMEM((16, D), jnp.float32),
      pltpu.SemaphoreType.DMA,
      pltpu.SemaphoreType.DMA,
      pltpu.SemaphoreType.DMA,
      pltpu.SemaphoreType.DMA,
      pltpu.SemaphoreType.DMA,
  ]
  if with_deg:
    scratch += [
        pltpu.VMEM_SHARED((N,), jnp.float32),
        pltpu.VMEM((128,), jnp.float32),
        pltpu.VMEM((640,), jnp.float32),
        pltpu.SemaphoreType.DMA,
        pltpu.SemaphoreType.DMA,
    ]

  def body(x_hbm, srcm, dstm, *rest):
    if with_deg:
      (agg_a, agg_b, deg_a, deg_b, agg_sh, src_v, dst_v, gbuf0, gbuf1, zbuf,
       gsem0, gsem1, ssem0, ssem1, zsem,
       deg_sh, ones_v, zb1, dsem0, dsem1) = rest
    else:
      (agg_a, agg_b, agg_sh, src_v, dst_v, gbuf0, gbuf1, zbuf,
       gsem0, gsem1, ssem0, ssem1, zsem) = rest
    gbufs, gsems, ssems = (gbuf0, gbuf1), (gsem0, gsem1), (ssem0, ssem1)
    if with_deg:
      dsems = (dsem0, dsem1)

    cid = lax.axis_index("c")
    sid = lax.axis_index("s")
    wid = cid * NS + sid

    zeros16 = jnp.zeros((16,), jnp.float32)

    def zstore(t, _):
      i = t // (D // 16)
      j = t % (D // 16)
      zbuf[i, pl.ds(j * 16, 16)] = zeros16
      return ()
    lax.fori_loop(0, 16 * (D // 16), zstore, ())

    def zero_strip(base, nrows):
      def zfire(q, _):
        pltpu.async_copy(zbuf, agg_sh.at[pl.ds(base + q * 16, 16)], zsem)
        return ()
      lax.fori_loop(0, nrows // 16, zfire, ())
      def zdrain(q, _):
        pltpu.make_async_copy(zbuf, agg_sh.at[pl.ds(base, 16)], zsem).wait()
        return ()
      lax.fori_loop(0, nrows // 16, zdrain, ())
    _per_tile_strip(sid, zero_strip)

    if with_deg:
      def zstore1(t, _):
        zb1[pl.ds(t * 16, 16)] = zeros16
        return ()
      lax.fori_loop(0, 640 // 16, zstore1, ())
      def ostore(t, _):
        ones_v[pl.ds(t * 16, 16)] = jnp.full((16,), 1.0, jnp.float32)
        return ()
      lax.fori_loop(0, 128 // 16, ostore, ())
      @pl.when(sid < 15)
      def _():
        pltpu.sync_copy(zb1, deg_sh.at[pl.ds(sid * 640, 640)])
      @pl.when(sid == 15)
      def _():
        pltpu.sync_copy(zb1.at[pl.ds(0, 400)], deg_sh.at[pl.ds(9600, 400)])

    plsc.subcore_barrier()

    def wait_gather(b):
      pltpu.make_async_copy(x_hbm.at[src_v.at[0]], gbufs[b], gsems[b]).wait()

    def wait_scatter(b):
      pltpu.make_async_copy(gbufs[b], agg_sh.at[dst_v.at[0]],
                            ssems[b]).wait()
      if with_deg:
        pltpu.make_async_copy(ones_v.at[pl.ds(0, K)], deg_sh.at[dst_v.at[0]],
                              dsems[b]).wait()

    def run_phase(p):
      pbase = wid * ROWS_W + p * HR
      pltpu.sync_copy(srcm.at[pl.ds(pbase, HR)], src_v)
      pltpu.sync_copy(dstm.at[pl.ds(pbase, HR)], dst_v)
      for b in range(2):
        pltpu.async_copy(x_hbm.at[src_v.at[b]], gbufs[b], gsems[b])

      def pair(jj, _):
        for b in range(2):
          j = 2 * jj + b
          wait_gather(b)
          pltpu.async_copy(gbufs[b], agg_sh.at[dst_v.at[j]], ssems[b],
                           add=True)
          if with_deg:
            pltpu.async_copy(ones_v.at[pl.ds(0, K)], deg_sh.at[dst_v.at[j]],
                             dsems[b], add=True)
        for b in range(2):
          j = 2 * jj + b
          @pl.when(j + 2 < HR)
          def _(b=b, j=j):
            wait_scatter(b)
            pltpu.async_copy(x_hbm.at[src_v.at[j + 2]], gbufs[b], gsems[b])
        return ()
      lax.fori_loop(0, HR // 2, pair, ())
      for b in range(2):
        wait_scatter(b)

    run_phase(0)
    run_phase(1)

    plsc.subcore_barrier()

    for c, agg_o in ((0, agg_a), (1, agg_b)):
      @pl.when(cid == c)
      def _(agg_o=agg_o):
        def drain_strip(base, nrows):
          pltpu.sync_copy(agg_sh.at[pl.ds(base, nrows)],
                          agg_o.at[pl.ds(base, nrows)])
        _per_tile_strip(sid, drain_strip)
    if with_deg:
      for c, deg_o in ((0, deg_a), (1, deg_b)):
        @pl.when(cid == c)
        def _(deg_o=deg_o):
          def deg_strip(base, nrows):
            pltpu.sync_copy(deg_sh.at[pl.ds(base, nrows)],
                            zb1.at[pl.ds(0, nrows)])
            pltpu.sync_copy(zb1.at[pl.ds(0, nrows)],
                            deg_o.at[pl.ds(base, nrows)])
          _per_tile_strip(sid, deg_strip)

  return pl.kernel(body, out_type=out_type, mesh=mesh,
                   scratch_types=scratch)


_agg128_deg = _build_agg(D_HID, with_deg=True)
_agg128 = _build_agg(D_HID, with_deg=False)


def _dense_relu(agg_a, agg_b, deg_a, deg_b, W, b):
  def body(aa, ab, da, db, w, bb, out):
    dinv = 1.0 / jnp.maximum(da[...] + db[...], 1.0)
    acc = (aa[...] + ab[...]) * dinv
    out[...] = jnp.maximum(
        jnp.dot(acc, w[...], preferred_element_type=jnp.float32) + bb[...],
        0.0)
  return pl.pallas_call(
      body,
      out_shape=jax.ShapeDtypeStruct((N, W.shape[1]), jnp.float32),
  )(agg_a, agg_b, deg_a, deg_b, W, b)


def _dense_final(agg_a, agg_b, deg_a, deg_b, W, b):
  def body(aa, ab, da, db, w, bb, out):
    dinv = 1.0 / jnp.maximum(da[...] + db[...], 1.0)
    acc = (aa[...] + ab[...]) * dinv
    out[...] = jnp.dot(acc, w[...], preferred_element_type=jnp.float32) + bb[...]
  return pl.pallas_call(
      body,
      out_shape=jax.ShapeDtypeStruct((N, W.shape[1]), jnp.float32),
  )(agg_a, agg_b, deg_a, deg_b, W, b)


def kernel(nodes, edge_index, W1, b1, W2, b2, W3, b3):
  ei = edge_index.astype(jnp.int32)
  srcm = ei[0].reshape(NROWS, K)
  dstm = ei[1].reshape(NROWS, K)
  b1r = b1.reshape(1, D_HID)
  b2r = b2.reshape(1, D_HID)
  b3r = b3.reshape(1, D_OUT)

  a1a, a1b, dega, degb = _agg128_deg(nodes, srcm, dstm)
  dega = dega.reshape(N, 1)
  degb = degb.reshape(N, 1)
  h1 = _dense_relu(a1a, a1b, dega, degb, W1, b1r)
  a2a, a2b = _agg128(h1, srcm, dstm)
  h2 = _dense_relu(a2a, a2b, dega, degb, W2, b2r)
  a3a, a3b = _agg128(h2, srcm, dstm)
  return _dense_final(a3a, a3b, dega, degb, W3, b3r)


# speedup vs baseline: 10.9563x; 1.0116x over previous
import functools

import jax
import jax.numpy as jnp
from jax import lax
from jax.experimental import pallas as pl
from jax.experimental.pallas import tpu as pltpu
from jax.experimental.pallas import tpu_sc as plsc

N = 10000
E = 320000
D_IN = 128
D_HID = 128
D_OUT = 16

K = 125
NROWS = E // K
NC, NS = 2, 16
NW = NC * NS
ROWS_W = NROWS // NW
HR = ROWS_W // 2
ZR = 640
ZLAST = N - 15 * ZR


def _per_tile_strip(sid, fn):
  @pl.when(sid < 15)
  def _():
    fn(sid * ZR, ZR)
  @pl.when(sid == 15)
  def _():
    fn(15 * ZR, ZLAST)


def _build_agg(D, with_deg):
  mesh = plsc.VectorSubcoreMesh(core_axis_name="c", subcore_axis_name="s")

  out_type = [jax.ShapeDtypeStruct((N, D), jnp.float32),
              jax.ShapeDtypeStruct((N, D), jnp.float32)]
  if with_deg:
    out_type += [jax.ShapeDtypeStruct((N,), jnp.float32),
                 jax.ShapeDtypeStruct((N,), jnp.float32)]

  scratch = [
      pltpu.VMEM_SHARED((N, D), jnp.float32),
      pltpu.VMEM((HR, K), jnp.int32),
      pltpu.VMEM((HR, K), jnp.int32),
      pltpu.VMEM((K, D), jnp.float32),
      pltpu.VMEM((K, D), jnp.float32),
      pltpu.VMEM((16, D), jnp.float32),
      pltpu.SemaphoreType.DMA,
      pltpu.SemaphoreType.DMA,
      pltpu.SemaphoreType.DMA,
      pltpu.SemaphoreType.DMA,
      pltpu.SemaphoreType.DMA,
  ]
  if with_deg:
    scratch += [
        pltpu.VMEM_SHARED((N,), jnp.float32),
        pltpu.VMEM((128,), jnp.float32),
        pltpu.VMEM((640,), jnp.float32),
        pltpu.SemaphoreType.DMA,
        pltpu.SemaphoreType.DMA,
    ]

  def body(x_hbm, srcm, dstm, *rest):
    if with_deg:
      (agg_a, agg_b, deg_a, deg_b, agg_sh, src_v, dst_v, gbuf0, gbuf1, zbuf,
       gsem0, gsem1, ssem0, ssem1, zsem,
       deg_sh, ones_v, zb1, dsem0, dsem1) = rest
    else:
      (agg_a, agg_b, agg_sh, src_v, dst_v, gbuf0, gbuf1, zbuf,
       gsem0, gsem1, ssem0, ssem1, zsem) = rest
    gbufs, gsems, ssems = (gbuf0, gbuf1), (gsem0, gsem1), (ssem0, ssem1)
    if with_deg:
      dsems = (dsem0, dsem1)

    cid = lax.axis_index("c")
    sid = lax.axis_index("s")
    wid = cid * NS + sid

    zeros16 = jnp.zeros((16,), jnp.float32)

    def zstore(t, _):
      i = t // (D // 16)
      j = t % (D // 16)
      zbuf[i, pl.ds(j * 16, 16)] = zeros16
      return ()
    lax.fori_loop(0, 16 * (D // 16), zstore, ())

    def zero_fire(base, nrows):
      def zfire(q, _):
        pltpu.async_copy(zbuf, agg_sh.at[pl.ds(base + q * 16, 16)], zsem)
        return ()
      lax.fori_loop(0, nrows // 16, zfire, ())
    def zero_drain(base, nrows):
      def zdrain(q, _):
        pltpu.make_async_copy(zbuf, agg_sh.at[pl.ds(base, 16)], zsem).wait()
        return ()
      lax.fori_loop(0, nrows // 16, zdrain, ())
    _per_tile_strip(sid, zero_fire)

    if with_deg:
      def zstore1(t, _):
        zb1[pl.ds(t * 16, 16)] = zeros16
        return ()
      lax.fori_loop(0, 640 // 16, zstore1, ())
      def ostore(t, _):
        ones_v[pl.ds(t * 16, 16)] = jnp.full((16,), 1.0, jnp.float32)
        return ()
      lax.fori_loop(0, 128 // 16, ostore, ())
      @pl.when(sid < 15)
      def _():
        pltpu.sync_copy(zb1, deg_sh.at[pl.ds(sid * 640, 640)])
      @pl.when(sid == 15)
      def _():
        pltpu.sync_copy(zb1.at[pl.ds(0, 400)], deg_sh.at[pl.ds(9600, 400)])

    def wait_gather(b):
      pltpu.make_async_copy(x_hbm.at[src_v.at[0]], gbufs[b], gsems[b]).wait()

    def wait_scatter(b):
      pltpu.make_async_copy(gbufs[b], agg_sh.at[dst_v.at[0]],
                            ssems[b]).wait()
      if with_deg:
        pltpu.make_async_copy(ones_v.at[pl.ds(0, K)], deg_sh.at[dst_v.at[0]],
                              dsems[b]).wait()

    def stage_and_prime(p):
      pbase = wid * ROWS_W + p * HR
      pltpu.sync_copy(srcm.at[pl.ds(pbase, HR)], src_v)
      pltpu.sync_copy(dstm.at[pl.ds(pbase, HR)], dst_v)
      for b in range(2):
        pltpu.async_copy(x_hbm.at[src_v.at[b]], gbufs[b], gsems[b])

    def run_ring():
      def pair(jj, _):
        for b in range(2):
          j = 2 * jj + b
          wait_gather(b)
          pltpu.async_copy(gbufs[b], agg_sh.at[dst_v.at[j]], ssems[b],
                           add=True)
          if with_deg:
            pltpu.async_copy(ones_v.at[pl.ds(0, K)], deg_sh.at[dst_v.at[j]],
                             dsems[b], add=True)
        for b in range(2):
          j = 2 * jj + b
          @pl.when(j + 2 < HR)
          def _(b=b, j=j):
            wait_scatter(b)
            pltpu.async_copy(x_hbm.at[src_v.at[j + 2]], gbufs[b], gsems[b])
        return ()
      lax.fori_loop(0, HR // 2, pair, ())
      for b in range(2):
        wait_scatter(b)

    stage_and_prime(0)
    _per_tile_strip(sid, zero_drain)
    plsc.subcore_barrier()
    run_ring()
    stage_and_prime(1)
    run_ring()

    plsc.subcore_barrier()

    for c, agg_o in ((0, agg_a), (1, agg_b)):
      @pl.when(cid == c)
      def _(agg_o=agg_o):
        def drain_strip(base, nrows):
          pltpu.sync_copy(agg_sh.at[pl.ds(base, nrows)],
                          agg_o.at[pl.ds(base, nrows)])
        _per_tile_strip(sid, drain_strip)
    if with_deg:
      for c, deg_o in ((0, deg_a), (1, deg_b)):
        @pl.when(cid == c)
        def _(deg_o=deg_o):
          def deg_strip(base, nrows):
            pltpu.sync_copy(deg_sh.at[pl.ds(base, nrows)],
                            zb1.at[pl.ds(0, nrows)])
            pltpu.sync_copy(zb1.at[pl.ds(0, nrows)],
                            deg_o.at[pl.ds(base, nrows)])
          _per_tile_strip(sid, deg_strip)

  return pl.kernel(body, out_type=out_type, mesh=mesh,
                   scratch_types=scratch)


_agg128_deg = _build_agg(D_HID, with_deg=True)
_agg128 = _build_agg(D_HID, with_deg=False)


def _dense_relu(agg_a, agg_b, deg_a, deg_b, W, b):
  def body(aa, ab, da, db, w, bb, out):
    dinv = 1.0 / jnp.maximum(da[...] + db[...], 1.0)
    acc = (aa[...] + ab[...]) * dinv
    out[...] = jnp.maximum(
        jnp.dot(acc, w[...], preferred_element_type=jnp.float32) + bb[...],
        0.0)
  return pl.pallas_call(
      body,
      out_shape=jax.ShapeDtypeStruct((N, W.shape[1]), jnp.float32),
  )(agg_a, agg_b, deg_a, deg_b, W, b)


def _dense_final(agg_a, agg_b, deg_a, deg_b, W, b):
  def body(aa, ab, da, db, w, bb, out):
    dinv = 1.0 / jnp.maximum(da[...] + db[...], 1.0)
    acc = (aa[...] + ab[...]) * dinv
    out[...] = jnp.dot(acc, w[...],
                       preferred_element_type=jnp.float32) + bb[...]
  return pl.pallas_call(
      body,
      out_shape=jax.ShapeDtypeStruct((N, W.shape[1]), jnp.float32),
  )(agg_a, agg_b, deg_a, deg_b, W, b)


def kernel(nodes, edge_index, W1, b1, W2, b2, W3, b3):
  ei = edge_index.astype(jnp.int32)
  srcm = ei[0].reshape(NROWS, K)
  dstm = ei[1].reshape(NROWS, K)
  b1r = b1.reshape(1, D_HID)
  b2r = b2.reshape(1, D_HID)
  b3r = b3.reshape(1, D_OUT)

  a1a, a1b, dega, degb = _agg128_deg(nodes, srcm, dstm)
  dega = dega.reshape(N, 1)
  degb = degb.reshape(N, 1)
  h1 = _dense_relu(a1a, a1b, dega, degb, W1, b1r)
  a2a, a2b = _agg128(h1, srcm, dstm)
  h2 = _dense_relu(a2a, a2b, dega, degb, W2, b2r)
  a3a, a3b = _agg128(h2, srcm, dstm)
  return _dense_final(a3a, a3b, dega, degb, W3, b3r)
